# baseline (speedup 1.0000x reference)
import functools

import jax
import jax.numpy as jnp
from jax import lax
from jax.experimental import pallas as pl
from jax.experimental.pallas import tpu as pltpu
from jax.experimental.pallas import tpu_sc as plsc

_OFFSET = 2


def kernel(input_ids, pos_table):
    bsz, seq_len = input_ids.shape
    dim = pos_table.shape[-1]

    info = plsc.get_sparse_core_info()
    num_cores, num_subcores = info.num_cores, info.num_subcores
    num_workers = num_cores * num_subcores
    rows_per_worker = seq_len // num_workers
    chunk_rows = 64
    n_chunks = rows_per_worker // chunk_rows
    chunk = chunk_rows * dim

    @functools.partial(
        pl.kernel,
        mesh=plsc.VectorSubcoreMesh(core_axis_name="c", subcore_axis_name="s"),
        out_type=jax.ShapeDtypeStruct((bsz * seq_len * dim,), jnp.float32),
        scratch_types=[
            pltpu.VMEM((chunk,), jnp.float32),
            pltpu.SemaphoreType.DMA,
            pltpu.SemaphoreType.DMA,
        ],
    )
    def pe_kernel(table_hbm, out_hbm, buf, in_sem, out_sem):
        wid = lax.axis_index("s") * num_cores + lax.axis_index("c")
        base = wid * rows_per_worker * dim
        for g in range(n_chunks):
            off = base + g * chunk
            pltpu.async_copy(
                table_hbm.at[pl.ds(off + _OFFSET * dim, chunk)], buf, in_sem
            ).wait()
            copies = [
                pltpu.async_copy(
                    buf, out_hbm.at[pl.ds(b * seq_len * dim + off, chunk)], out_sem
                )
                for b in range(bsz)
            ]
            for c in copies:
                c.wait()

    out_flat = pe_kernel(pos_table.reshape(-1))
    return out_flat.reshape(bsz, seq_len, dim)
